# baseline (speedup 1.0000x reference)
import functools

import jax
import jax.numpy as jnp
from jax import lax
from jax.experimental import pallas as pl
from jax.experimental.pallas import tpu as pltpu
from jax.experimental.pallas import tpu_sc as plsc

TOK = 32000
B_TOTAL = 8192
D = 1024
NC = 2
NS = 16
NW = NC * NS
BPW = B_TOTAL // NW
CHUNKS = (40, 40, 40, 40, 40, 40, 16)
OFFS = (0, 40, 80, 120, 160, 200, 240)
CMAX = max(CHUNKS)
NCHUNK = len(CHUNKS)
NBUF = 3

_mesh = plsc.VectorSubcoreMesh(core_axis_name="c", subcore_axis_name="s")


@functools.partial(
    pl.kernel,
    mesh=_mesh,
    out_type=jax.ShapeDtypeStruct((B_TOTAL, D), jnp.float32),
    scratch_types=[
        pltpu.VMEM((BPW + 16,), jnp.int32),
        pltpu.VMEM((BPW,), jnp.int32),
        pltpu.VMEM((NBUF, CMAX, D), jnp.float32),
        pltpu.SemaphoreType.DMA,
        pltpu.SemaphoreType.DMA,
        pltpu.SemaphoreType.DMA,
        pltpu.SemaphoreType.DMA,
        pltpu.SemaphoreType.DMA,
        pltpu.SemaphoreType.DMA,
    ],
)
def _emb_lookup(idx_hbm, main_hbm, prompt_hbm, out_hbm, idx_v, safe_v, rows_v,
                g0, g1, g2, w0, w1, w2):
    gsem = (g0, g1, g2)
    wsem = (w0, w1, w2)
    wid = lax.axis_index("s") * NC + lax.axis_index("c")
    base = wid * BPW
    pltpu.sync_copy(idx_hbm.at[pl.ds(base, BPW)], idx_v.at[pl.ds(0, BPW)])

    for g in range(BPW // 16):
        v = idx_v[pl.ds(g * 16, 16)]
        safe_v[pl.ds(g * 16, 16)] = jnp.where(v >= TOK, 0, v)

    def gather(c):
        b = c % NBUF
        return pltpu.async_copy(
            main_hbm.at[safe_v.at[pl.ds(OFFS[c], CHUNKS[c])]],
            rows_v.at[b].at[pl.ds(0, CHUNKS[c])],
            gsem[b],
        )

    gh = {}
    wh = {}
    for c in range(min(NBUF, NCHUNK)):
        gh[c] = gather(c)

    for c in range(NCHUNK):
        b = c % NBUF
        gh[c].wait()

        def body(i, carry, c=c, b=b):
            s = idx_v[pl.ds(OFFS[c] + i, 16)][0]

            @pl.when(s >= TOK)
            def _():
                pltpu.sync_copy(
                    prompt_hbm.at[pl.ds(s - TOK, 1)],
                    rows_v.at[b].at[pl.ds(i, 1)],
                )

            return carry

        lax.fori_loop(0, CHUNKS[c], body, 0)

        wh[c] = pltpu.async_copy(
            rows_v.at[b].at[pl.ds(0, CHUNKS[c])],
            out_hbm.at[pl.ds(base + OFFS[c], CHUNKS[c])],
            wsem[b],
        )
        nxt = c + NBUF
        if nxt < NCHUNK:
            wh[nxt - NBUF].wait()
            gh[nxt] = gather(nxt)

    for c in range(max(0, NCHUNK - NBUF), NCHUNK):
        wh[c].wait()


def kernel(input, main_embeddings, prompt_embeddings):
    idx = input.reshape(-1).astype(jnp.int32)
    out = _emb_lookup(idx, main_embeddings, prompt_embeddings)
    return out.reshape(input.shape + (D,))


# speedup vs baseline: 1.0215x; 1.0215x over previous
import functools

import jax
import jax.numpy as jnp
from jax import lax
from jax.experimental import pallas as pl
from jax.experimental.pallas import tpu as pltpu
from jax.experimental.pallas import tpu_sc as plsc

TOK = 32000
B_TOTAL = 8192
D = 1024
NC = 2
NS = 16
NW = NC * NS
BPW = B_TOTAL // NW
CHUNK = 32
NCHUNK = BPW // CHUNK
NBUF = 3

_mesh = plsc.VectorSubcoreMesh(core_axis_name="c", subcore_axis_name="s")


@functools.partial(
    pl.kernel,
    mesh=_mesh,
    out_type=jax.ShapeDtypeStruct((B_TOTAL, D), jnp.float32),
    scratch_types=[
        pltpu.VMEM((BPW + 16,), jnp.int32),
        pltpu.VMEM((BPW,), jnp.int32),
        pltpu.VMEM((NBUF, CHUNK, D), jnp.float32),
        pltpu.SemaphoreType.DMA,
        pltpu.SemaphoreType.DMA,
        pltpu.SemaphoreType.DMA,
        pltpu.SemaphoreType.DMA,
        pltpu.SemaphoreType.DMA,
        pltpu.SemaphoreType.DMA,
    ],
)
def _emb_lookup(idx_hbm, main_hbm, prompt_hbm, out_hbm, idx_v, safe_v, rows_v,
                g0, g1, g2, w0, w1, w2):
    gsem = (g0, g1, g2)
    wsem = (w0, w1, w2)
    wid = lax.axis_index("s") * NC + lax.axis_index("c")
    base = wid * BPW
    pltpu.sync_copy(idx_hbm.at[pl.ds(base, BPW)], idx_v.at[pl.ds(0, BPW)])

    for g in range(BPW // 16):
        v = idx_v[pl.ds(g * 16, 16)]
        safe_v[pl.ds(g * 16, 16)] = jnp.where(v >= TOK, 0, v)

    def gather(c):
        b = c % NBUF
        return pltpu.async_copy(
            main_hbm.at[safe_v.at[pl.ds(c * CHUNK, CHUNK)]], rows_v.at[b], gsem[b]
        )

    gh = {}
    wh = {}
    for c in range(min(NBUF, NCHUNK)):
        gh[c] = gather(c)

    for c in range(NCHUNK):
        b = c % NBUF
        gh[c].wait()

        def body(i, carry, c=c, b=b):
            s = idx_v[pl.ds(c * CHUNK + i, 16)][0]

            @pl.when(s >= TOK)
            def _():
                pltpu.sync_copy(
                    prompt_hbm.at[pl.ds(s - TOK, 1)],
                    rows_v.at[b].at[pl.ds(i, 1)],
                )

            return carry

        lax.fori_loop(0, CHUNK, body, 0)

        wh[c] = pltpu.async_copy(
            rows_v.at[b], out_hbm.at[pl.ds(base + c * CHUNK, CHUNK)], wsem[b]
        )
        nxt = c + NBUF
        if nxt < NCHUNK:
            wh[nxt - NBUF].wait()
            gh[nxt] = gather(nxt)

    for c in range(max(0, NCHUNK - NBUF), NCHUNK):
        wh[c].wait()


def kernel(input, main_embeddings, prompt_embeddings):
    idx = input.reshape(-1).astype(jnp.int32)
    out = _emb_lookup(idx, main_embeddings, prompt_embeddings)
    return out.reshape(input.shape + (D,))


# speedup vs baseline: 1.0278x; 1.0062x over previous
import functools

import jax
import jax.numpy as jnp
from jax import lax
from jax.experimental import pallas as pl
from jax.experimental.pallas import tpu as pltpu
from jax.experimental.pallas import tpu_sc as plsc

TOK = 32000
B_TOTAL = 8192
D = 1024
NC = 2
NS = 16
NW = NC * NS
BPW = B_TOTAL // NW
CHUNK = 32
NCHUNK = BPW // CHUNK
NBUF = 3

_mesh = plsc.VectorSubcoreMesh(core_axis_name="c", subcore_axis_name="s")


@functools.partial(
    pl.kernel,
    mesh=_mesh,
    out_type=jax.ShapeDtypeStruct((B_TOTAL, D), jnp.float32),
    scratch_types=[
        pltpu.VMEM((BPW + 16,), jnp.int32),
        pltpu.VMEM((BPW,), jnp.int32),
        pltpu.VMEM((NBUF, CHUNK, D), jnp.float32),
        pltpu.SemaphoreType.DMA,
        pltpu.SemaphoreType.DMA,
        pltpu.SemaphoreType.DMA,
        pltpu.SemaphoreType.DMA,
        pltpu.SemaphoreType.DMA,
        pltpu.SemaphoreType.DMA,
    ],
)
def _emb_lookup(idx_hbm, main_hbm, prompt_hbm, out_hbm, idx_v, safe_v, rows_v,
                g0, g1, g2, w0, w1, w2):
    gsem = (g0, g1, g2)
    wsem = (w0, w1, w2)
    wid = lax.axis_index("c") * NS + lax.axis_index("s")
    base = wid * BPW
    pltpu.sync_copy(idx_hbm.at[pl.ds(base, BPW)], idx_v.at[pl.ds(0, BPW)])

    for g in range(BPW // 16):
        v = idx_v[pl.ds(g * 16, 16)]
        safe_v[pl.ds(g * 16, 16)] = jnp.where(v >= TOK, 0, v)

    def gather(c):
        b = c % NBUF
        return pltpu.async_copy(
            main_hbm.at[safe_v.at[pl.ds(c * CHUNK, CHUNK)]], rows_v.at[b], gsem[b]
        )

    gh = {}
    wh = {}
    for c in range(min(NBUF, NCHUNK)):
        gh[c] = gather(c)

    for c in range(NCHUNK):
        b = c % NBUF
        gh[c].wait()

        def body(i, carry, c=c, b=b):
            s = idx_v[pl.ds(c * CHUNK + i, 16)][0]

            @pl.when(s >= TOK)
            def _():
                pltpu.sync_copy(
                    prompt_hbm.at[pl.ds(s - TOK, 1)],
                    rows_v.at[b].at[pl.ds(i, 1)],
                )

            return carry

        lax.fori_loop(0, CHUNK, body, 0)

        wh[c] = pltpu.async_copy(
            rows_v.at[b], out_hbm.at[pl.ds(base + c * CHUNK, CHUNK)], wsem[b]
        )
        nxt = c + NBUF
        if nxt < NCHUNK:
            wh[nxt - NBUF].wait()
            gh[nxt] = gather(nxt)

    for c in range(max(0, NCHUNK - NBUF), NCHUNK):
        wh[c].wait()


def kernel(input, main_embeddings, prompt_embeddings):
    idx = input.reshape(-1).astype(jnp.int32)
    out = _emb_lookup(idx, main_embeddings, prompt_embeddings)
    return out.reshape(input.shape + (D,))
